# Initial kernel scaffold; baseline (speedup 1.0000x reference)
import jax
import jax.numpy as jnp
from jax.experimental import pallas as pl


def kernel(x, edge_index, W1_l, b1_l, W1_r, W2_l, b2_l, W2_r):
    raise NotImplementedError("write your pallas kernel here")


# speedup vs baseline: 4.9981x; 4.9981x over previous
import functools

import jax
import jax.numpy as jnp
from jax import lax
from jax.experimental import pallas as pl
from jax.experimental.pallas import tpu as pltpu
from jax.experimental.pallas import tpu_sc as plsc

N = 10000
D = 128
E = 320000

NC = 2
NS = 16
NW = NC * NS
PER_CORE = E // NC
PER_TILE = PER_CORE // NS
CH = 80
NCHUNK = PER_TILE // CH
NPAD = 10240
ROWS_PER_TILE = NPAD // NS
L = 16

_mesh = plsc.VectorSubcoreMesh(core_axis_name="c", subcore_axis_name="s")


@functools.partial(
    pl.kernel,
    out_type=[
        jax.ShapeDtypeStruct((NC, NPAD, D), jnp.float32),
        jax.ShapeDtypeStruct((NC, NS, NPAD), jnp.float32),
    ],
    mesh=_mesh,
    compiler_params=pltpu.CompilerParams(needs_layout_passes=False),
    scratch_types=[
        pltpu.VMEM((CH,), jnp.int32),
        pltpu.VMEM((CH,), jnp.int32),
        pltpu.VMEM((CH, D), jnp.float32),
        pltpu.VMEM((NPAD,), jnp.float32),
        pltpu.VMEM_SHARED((NPAD, D), jnp.float32),
        pltpu.SemaphoreType.DMA,
    ],
)
def _agg_counts(x_hbm, src_hbm, dst_hbm, zrow_hbm,
                acc_out, cnt_out,
                src_v, dst_v, rows_v, cnt_v, acc_sh, sem):
    c = lax.axis_index("c")
    s = lax.axis_index("s")
    ebase = c * PER_CORE + s * PER_TILE
    rbase = s * ROWS_PER_TILE
    pltpu.sync_copy(zrow_hbm.at[pl.ds(rbase, ROWS_PER_TILE)],
                    acc_sh.at[pl.ds(rbase, ROWS_PER_TILE)])

    zeros = jnp.zeros((L,), jnp.float32)

    def zbody(j, carry):
        cnt_v[pl.ds(j * L, L)] = zeros
        return carry

    lax.fori_loop(0, NPAD // L, zbody, 0)
    plsc.subcore_barrier()

    ones = jnp.ones((L,), jnp.float32)

    def body(i, carry):
        off = ebase + i * CH
        pltpu.sync_copy(src_hbm.at[pl.ds(off, CH)], src_v)
        pltpu.sync_copy(dst_hbm.at[pl.ds(off, CH)], dst_v)
        pltpu.async_copy(x_hbm.at[src_v], rows_v, sem).wait()
        pltpu.sync_copy(rows_v, acc_sh.at[dst_v], add=True)
        for k in range(CH // L):
            idx = dst_v[pl.ds(k * L, L)]
            plsc.addupdate_scatter(cnt_v, [idx], ones)
        return carry

    lax.fori_loop(0, NCHUNK, body, 0)
    plsc.subcore_barrier()
    pltpu.sync_copy(acc_sh.at[pl.ds(rbase, ROWS_PER_TILE)],
                    acc_out.at[c, pl.ds(rbase, ROWS_PER_TILE)])
    pltpu.sync_copy(cnt_v, cnt_out.at[c, s])


@functools.partial(
    pl.kernel,
    out_type=jax.ShapeDtypeStruct((NC, NPAD, D), jnp.float32),
    mesh=_mesh,
    compiler_params=pltpu.CompilerParams(needs_layout_passes=False),
    scratch_types=[
        pltpu.VMEM((CH,), jnp.int32),
        pltpu.VMEM((CH,), jnp.int32),
        pltpu.VMEM((CH, D), jnp.float32),
        pltpu.VMEM_SHARED((NPAD, D), jnp.float32),
        pltpu.SemaphoreType.DMA,
    ],
)
def _agg(x_hbm, src_hbm, dst_hbm, zrow_hbm,
         acc_out,
         src_v, dst_v, rows_v, acc_sh, sem):
    c = lax.axis_index("c")
    s = lax.axis_index("s")
    ebase = c * PER_CORE + s * PER_TILE
    rbase = s * ROWS_PER_TILE
    pltpu.sync_copy(zrow_hbm.at[pl.ds(rbase, ROWS_PER_TILE)],
                    acc_sh.at[pl.ds(rbase, ROWS_PER_TILE)])
    plsc.subcore_barrier()

    def body(i, carry):
        off = ebase + i * CH
        pltpu.sync_copy(src_hbm.at[pl.ds(off, CH)], src_v)
        pltpu.sync_copy(dst_hbm.at[pl.ds(off, CH)], dst_v)
        pltpu.async_copy(x_hbm.at[src_v], rows_v, sem).wait()
        pltpu.sync_copy(rows_v, acc_sh.at[dst_v], add=True)
        return carry

    lax.fori_loop(0, NCHUNK, body, 0)
    plsc.subcore_barrier()
    pltpu.sync_copy(acc_sh.at[pl.ds(rbase, ROWS_PER_TILE)],
                    acc_out.at[c, pl.ds(rbase, ROWS_PER_TILE)])


def _dense_body(acc_ref, cnt_ref, x_ref, wl_ref, b_ref, wr_ref, o_ref):
    ssum = acc_ref[0] + acc_ref[1]
    cnt = jnp.sum(cnt_ref[...], axis=0)[:, None]
    mean = ssum / jnp.maximum(cnt, 1.0)
    z = (jnp.dot(mean, wl_ref[...], preferred_element_type=jnp.float32,
                 precision=lax.Precision.HIGHEST)
         + b_ref[...]
         + jnp.dot(x_ref[...], wr_ref[...], preferred_element_type=jnp.float32,
                   precision=lax.Precision.HIGHEST))
    h = jax.nn.sigmoid(z)
    nrm = jnp.sqrt(jnp.sum(h * h, axis=1, keepdims=True))
    o_ref[...] = h / jnp.maximum(nrm, 1e-12)


def _make_dense(dout, br=1024):
    return pl.pallas_call(
        _dense_body,
        grid=(NPAD // br,),
        in_specs=[
            pl.BlockSpec((NC, br, D), lambda i: (0, i, 0)),
            pl.BlockSpec((NW, br), lambda i: (0, i)),
            pl.BlockSpec((br, D), lambda i: (i, 0)),
            pl.BlockSpec((D, dout), lambda i: (0, 0)),
            pl.BlockSpec((1, dout), lambda i: (0, 0)),
            pl.BlockSpec((D, dout), lambda i: (0, 0)),
        ],
        out_specs=pl.BlockSpec((br, dout), lambda i: (i, 0)),
        out_shape=jax.ShapeDtypeStruct((NPAD, dout), jnp.float32),
    )


_dense128 = _make_dense(128)
_dense256 = _make_dense(256)


@jax.jit
def kernel(x, edge_index, W1_l, b1_l, W1_r, W2_l, b2_l, W2_r):
    src = edge_index[0].astype(jnp.int32)
    dst = edge_index[1].astype(jnp.int32)
    zrow = jnp.zeros((NPAD, D), jnp.float32)

    xp = jnp.zeros((NPAD, D), jnp.float32).at[:N].set(x)

    acc1, cnt3 = _agg_counts(x, src, dst, zrow)
    cnt = cnt3.reshape(NW, NPAD)
    h1 = _dense128(acc1, cnt, xp, W1_l.T, b1_l[None, :], W1_r.T)
    acc2 = _agg(h1, src, dst, zrow)
    h2 = _dense256(acc2, cnt, h1, W2_l.T, b2_l[None, :], W2_r.T)
    return h2[:N]
